# baseline (speedup 1.0000x reference)
import functools

import jax
import jax.numpy as jnp
from jax import lax
from jax.experimental import pallas as pl
from jax.experimental.pallas import tpu as pltpu
from jax.experimental.pallas import tpu_sc as plsc

D = 64
F = 2 * D + 1
FP = 144
L = 16
EPS = 1e-5
NC, NS = 2, 16
NW = NC * NS
C = 128


def _sc_embed(xh, xw, gvec, h_table, w_table, pos_pad, gamma_pad, beta_pad, T, N):
    per_w = N // NW
    chunks = per_w // C
    mesh = plsc.VectorSubcoreMesh(core_axis_name="c", subcore_axis_name="s")

    @functools.partial(
        pl.kernel,
        mesh=mesh,
        out_type=jax.ShapeDtypeStruct((N * F,), jnp.float32),
        compiler_params=pltpu.CompilerParams(needs_layout_passes=False,
                                             use_tc_tiling_on_sc=False),
        scratch_types=[
            pltpu.VMEM((C,), jnp.float32),
            pltpu.VMEM((C,), jnp.float32),
            pltpu.VMEM((C,), jnp.int32),
            pltpu.VMEM((C,), jnp.int32),
            pltpu.VMEM((C, D), jnp.float32),
            pltpu.VMEM((C, D), jnp.float32),
            pltpu.VMEM((C * F + L,), jnp.float32),
            pltpu.VMEM((T * FP,), jnp.float32),
            pltpu.VMEM((FP,), jnp.float32),
            pltpu.VMEM((FP,), jnp.float32),
            pltpu.VMEM((2 * L,), jnp.float32),
            pltpu.SemaphoreType.DMA,
            pltpu.SemaphoreType.DMA,
        ],
    )
    def k(xh_hbm, xw_hbm, g_hbm, h_hbm, w_hbm, pos_hbm, gam_hbm, bet_hbm, out_hbm,
          xh_v, xw_v, ih_v, iw_v, hr_v, wr_v, ob_v, pos_v, gam_v, bet_v, g_v,
          sem1, sem2):
        wid = lax.axis_index("s") * NC + lax.axis_index("c")
        base = wid * per_w
        pltpu.sync_copy(pos_hbm, pos_v)
        pltpu.sync_copy(gam_hbm, gam_v)
        pltpu.sync_copy(bet_hbm, bet_v)
        pltpu.sync_copy(g_hbm, g_v)

        iota = lax.iota(jnp.int32, L)
        g0v = g_v[pl.ds(0, L)]
        g1v = g_v[pl.ds(L, L)]
        gks = [gam_v[pl.ds(kk * L, L)] for kk in range(8)]
        bks = [bet_v[pl.ds(kk * L, L)] for kk in range(8)]
        gam128 = gam_v[pl.ds(2 * D, L)][0]
        bet128 = bet_v[pl.ds(2 * D, L)][0]
        inv_f = jnp.float32(1.0 / F)

        def chunk_body(c, carry):
            cbase = pl.multiple_of(base + c * C, C)
            pltpu.sync_copy(xh_hbm.at[pl.ds(cbase, C)], xh_v)
            pltpu.sync_copy(xw_hbm.at[pl.ds(cbase, C)], xw_v)

            def idx_body(i, carry2):
                xv = xh_v[pl.ds(i * L, L)]
                ih_v[pl.ds(i * L, L)] = ((xv + 1.0) / g0v).astype(jnp.int32)
                wv = xw_v[pl.ds(i * L, L)]
                iw_v[pl.ds(i * L, L)] = ((wv + 1.0) / g1v).astype(jnp.int32)
                return carry2

            lax.fori_loop(0, C // L, idx_body, 0)
            cp1 = pltpu.async_copy(h_hbm.at[ih_v], hr_v, sem1)
            cp2 = pltpu.async_copy(w_hbm.at[iw_v], wr_v, sem2)
            cp1.wait()
            cp2.wait()

            def group_body(gi, carry2):
                gp = gi * L
                xg = xh_v[pl.ds(gp, L)]
                u_vec = jnp.where(xg > 1.0, jnp.float32(1.0), jnp.float32(0.0))
                t_vec = lax.rem(cbase + gp + iota, T)
                for lane in range(L):
                    p = gp + lane
                    po = t_vec[lane] * FP
                    e = [hr_v[p, pl.ds(kk * L, L)] + pos_v[pl.ds(po + kk * L, L)]
                         for kk in range(4)]
                    e += [wr_v[p, pl.ds(kk * L, L)]
                          + pos_v[pl.ds(po + D + kk * L, L)] for kk in range(4)]
                    e8 = u_vec[lane] + pos_v[pl.ds(po + 2 * D, L)][0]
                    sv = ((e[0] + e[1]) + (e[2] + e[3])) + ((e[4] + e[5]) + (e[6] + e[7]))
                    q = [ei * ei for ei in e]
                    qv = ((q[0] + q[1]) + (q[2] + q[3])) + ((q[4] + q[5]) + (q[6] + q[7]))
                    ssum = jnp.sum(sv) + e8
                    qsum = jnp.sum(qv) + e8 * e8
                    mean = ssum * inv_f
                    var = qsum * inv_f - mean * mean
                    vv = jnp.full((L,), var + EPS, jnp.float32)
                    ii = plsc.bitcast(vv, jnp.int32)
                    ii = 0x5F3759DF - lax.shift_right_logical(ii, 1)
                    y = plsc.bitcast(ii, jnp.float32)
                    for _ in range(3):
                        y = y * (1.5 - 0.5 * vv * y * y)
                    mv = jnp.full((L,), mean, jnp.float32)
                    ob_base = p * F
                    for kk in range(8):
                        ob_v[pl.ds(ob_base + kk * L, L)] = (e[kk] - mv) * y * gks[kk] + bks[kk]
                    f8 = jnp.full((L,), e8 - mean, jnp.float32) * y * gam128 + bet128
                    ob_v[pl.ds(ob_base + 2 * D, L)] = f8
                return carry2

            lax.fori_loop(0, C // L, group_body, 0)
            pltpu.sync_copy(ob_v.at[pl.ds(0, C * F)],
                            out_hbm.at[pl.ds(pl.multiple_of(cbase * F, C), C * F)])
            return carry

        lax.fori_loop(0, chunks, chunk_body, 0)

    return k(xh, xw, gvec, h_table, w_table, pos_pad, gamma_pad, beta_pad)


def kernel(x, grid_size_tensor, h_table, w_table, pos_table, gamma, beta):
    B, T, _ = x.shape
    N = B * T
    xh = x[:, :, 0].reshape(N)
    xw = x[:, :, 1].reshape(N)
    gvec = jnp.concatenate([
        jnp.broadcast_to(grid_size_tensor[0], (L,)),
        jnp.broadcast_to(grid_size_tensor[1], (L,)),
    ])
    pos_pad = jnp.pad(pos_table, ((0, 0), (0, FP - F))).reshape(T * FP)
    gamma_pad = jnp.pad(gamma, (0, FP - F))
    beta_pad = jnp.pad(beta, (0, FP - F))
    out = _sc_embed(xh, xw, gvec, h_table, w_table, pos_pad, gamma_pad,
                    beta_pad, T, N)
    return out.reshape(B, T, F)
